# baseline (speedup 1.0000x reference)
import jax
import jax.numpy as jnp
from jax import lax
from jax.experimental import pallas as pl
from jax.experimental.pallas import tpu as pltpu

D_MODEL = 2048
HIDDEN = 1024
NUM_EXPERTS = 64
N_TOKENS = 16384
BLK = 1024
N_BLOCKS = N_TOKENS // BLK
LOAD_BALANCE_WEIGHT = 0.05
_INV_SQRT2 = 0.7071067811865476


def _router_kernel(x_ref, w1_ref, b1_ref, w2_ref, b2_ref,
                   oh_ref, loss_ref, h_ref, cnt_ref):
    i = pl.program_id(0)

    @pl.when(i == 0)
    def _init():
        cnt_ref[...] = jnp.zeros_like(cnt_ref)

    slot = lax.rem(i, 2)
    h_ref[slot] = lax.dot_general(
        x_ref[...], w1_ref[...], (((1,), (1,)), ((), ())),
        preferred_element_type=jnp.float32)

    h = h_ref[lax.rem(i + 1, 2)] + b1_ref[...]
    h = 0.5 * h * (1.0 + lax.erf(h * _INV_SQRT2))
    logits = lax.dot_general(h, w2_ref[...], (((1,), (1,)), ((), ())),
                             preferred_element_type=jnp.float32)
    logits = logits + b2_ref[...]
    m = jnp.max(logits, axis=1, keepdims=True)
    col = lax.broadcasted_iota(jnp.int32, logits.shape, 1)
    first = jnp.min(jnp.where(logits == m, col, NUM_EXPERTS),
                    axis=1, keepdims=True)
    oh = (col == first).astype(jnp.float32)
    oh_ref[...] = oh
    gate = jnp.where(i > 0, 1.0, 0.0).astype(jnp.float32)
    cnt_ref[...] += gate * jnp.sum(oh, axis=0, keepdims=True)

    @pl.when(i == N_BLOCKS)
    def _finalize():
        p = cnt_ref[...] * (1.0 / N_TOKENS)
        u = 1.0 / NUM_EXPERTS
        terms = u * (jnp.log(u) - jnp.log(p + 1e-10))
        kl = jnp.sum(terms, axis=1, keepdims=True) / NUM_EXPERTS
        loss_ref[...] = kl * LOAD_BALANCE_WEIGHT


def kernel(x, W1, b1, W2, b2):
    oh, loss = pl.pallas_call(
        _router_kernel,
        grid=(N_BLOCKS + 1,),
        in_specs=[
            pl.BlockSpec((BLK, D_MODEL),
                         lambda i: (jnp.minimum(i, N_BLOCKS - 1), 0)),
            pl.BlockSpec((HIDDEN, D_MODEL), lambda i: (0, 0)),
            pl.BlockSpec((1, HIDDEN), lambda i: (0, 0)),
            pl.BlockSpec((NUM_EXPERTS, HIDDEN), lambda i: (0, 0)),
            pl.BlockSpec((1, NUM_EXPERTS), lambda i: (0, 0)),
        ],
        out_specs=[
            pl.BlockSpec((BLK, NUM_EXPERTS),
                         lambda i: (jnp.maximum(i - 1, 0), 0)),
            pl.BlockSpec((1, 1), lambda i: (0, 0)),
        ],
        out_shape=[
            jax.ShapeDtypeStruct((N_TOKENS, NUM_EXPERTS), jnp.float32),
            jax.ShapeDtypeStruct((1, 1), jnp.float32),
        ],
        scratch_shapes=[
            pltpu.VMEM((2, BLK, HIDDEN), jnp.float32),
            pltpu.VMEM((1, NUM_EXPERTS), jnp.float32),
        ],
    )(x, W1, b1.reshape(1, HIDDEN), W2, b2.reshape(1, NUM_EXPERTS))
    return oh, loss[0, 0]


# speedup vs baseline: 1.2009x; 1.2009x over previous
import jax
import jax.numpy as jnp
from jax import lax
from jax.experimental import pallas as pl
from jax.experimental.pallas import tpu as pltpu

D_MODEL = 2048
HIDDEN = 1024
NUM_EXPERTS = 64
N_TOKENS = 16384
QTR = 512
BLK = 4 * QTR
LOAD_BALANCE_WEIGHT = 0.05
_INV_SQRT2 = 0.7071067811865476


def _router_block(x, w1, b1, w2, b2):
    h = lax.dot_general(x, w1, (((1,), (1,)), ((), ())),
                        preferred_element_type=jnp.float32)
    h = h + b1
    h = 0.5 * h * (1.0 + lax.erf(h * _INV_SQRT2))
    logits = lax.dot_general(h, w2, (((1,), (1,)), ((), ())),
                             preferred_element_type=jnp.float32)
    logits = logits + b2
    m = jnp.max(logits, axis=1, keepdims=True)
    col = lax.broadcasted_iota(jnp.int32, logits.shape, 1)
    first = jnp.min(jnp.where(logits == m, col, NUM_EXPERTS),
                    axis=1, keepdims=True)
    return (col == first).astype(jnp.float32)


def _router_kernel(xa_ref, xb_ref, xc_ref, xd_ref, w1_ref, b1_ref, w2_ref,
                   b2_ref, oh_ref, loss_ref, cnt_ref):
    i = pl.program_id(0)
    n_blocks = pl.num_programs(0)

    w1 = w1_ref[...]
    b1 = b1_ref[...]
    w2 = w2_ref[...]
    b2 = b2_ref[...]
    total = None
    for q, x_ref in enumerate((xa_ref, xb_ref, xc_ref, xd_ref)):
        oh_q = _router_block(x_ref[...], w1, b1, w2, b2)
        oh_ref[q * QTR:(q + 1) * QTR, :] = oh_q
        s = jnp.sum(oh_q, axis=0, keepdims=True)
        total = s if total is None else total + s

    @pl.when(i == 0)
    def _init():
        cnt_ref[...] = jnp.zeros_like(cnt_ref)

    cnt_ref[...] += total

    @pl.when(i == n_blocks - 1)
    def _finalize():
        p = cnt_ref[...] * (1.0 / N_TOKENS)
        u = 1.0 / NUM_EXPERTS
        terms = u * (jnp.log(u) - jnp.log(p + 1e-10))
        kl = jnp.sum(terms, axis=1, keepdims=True) / NUM_EXPERTS
        loss_ref[...] = kl * LOAD_BALANCE_WEIGHT


def kernel(x, W1, b1, W2, b2):
    grid = N_TOKENS // BLK
    oh, loss = pl.pallas_call(
        _router_kernel,
        grid=(grid,),
        in_specs=[
            pl.BlockSpec((QTR, D_MODEL), lambda i: (4 * i, 0)),
            pl.BlockSpec((QTR, D_MODEL), lambda i: (4 * i + 1, 0)),
            pl.BlockSpec((QTR, D_MODEL), lambda i: (4 * i + 2, 0)),
            pl.BlockSpec((QTR, D_MODEL), lambda i: (4 * i + 3, 0)),
            pl.BlockSpec((HIDDEN, D_MODEL), lambda i: (0, 0)),
            pl.BlockSpec((1, HIDDEN), lambda i: (0, 0)),
            pl.BlockSpec((NUM_EXPERTS, HIDDEN), lambda i: (0, 0)),
            pl.BlockSpec((1, NUM_EXPERTS), lambda i: (0, 0)),
        ],
        out_specs=[
            pl.BlockSpec((BLK, NUM_EXPERTS), lambda i: (i, 0)),
            pl.BlockSpec((1, 1), lambda i: (0, 0)),
        ],
        out_shape=[
            jax.ShapeDtypeStruct((N_TOKENS, NUM_EXPERTS), jnp.float32),
            jax.ShapeDtypeStruct((1, 1), jnp.float32),
        ],
        scratch_shapes=[pltpu.VMEM((1, NUM_EXPERTS), jnp.float32)],
    )(x, x, x, x, W1, b1.reshape(1, HIDDEN), W2, b2.reshape(1, NUM_EXPERTS))
    return oh, loss[0, 0]
